# baseline (speedup 1.0000x reference)
import functools

import jax
import jax.numpy as jnp
from jax import lax
from jax.experimental import pallas as pl
from jax.experimental.pallas import tpu as pltpu
from jax.experimental.pallas import tpu_sc as plsc

_T = 512
_TA = 4
_F = 1024
_NW = 32
_CH = 16


def _ffn_compute(x_ref, w1_ref, w2_ref, b1_ref, b2_ref, out_ref):
    f = pl.program_id(1)
    nf = pl.num_programs(1)
    h = jnp.dot(x_ref[...], w1_ref[0], preferred_element_type=jnp.float32)
    h = jnp.maximum(h + b1_ref[0, 0, 0, :][None, :], 0.0)
    p = jnp.dot(h, w2_ref[0], preferred_element_type=jnp.float32)

    @pl.when(f == 0)
    def _():
        out_ref[...] = p

    @pl.when(f > 0)
    def _():
        out_ref[...] += p

    @pl.when(f == nf - 1)
    def _():
        out_ref[...] += b2_ref[0, 0, :][None, :]


def _ffn_body_a(emap_ref, x_ref, w1_ref, w2_ref, b1_ref, b2_ref, out_ref):
    _ffn_compute(x_ref, w1_ref, w2_ref, b1_ref, b2_ref, out_ref)


def _ffn_body_b(emap_ref, x_ref, w1_ref, w2_ref, b1_ref, b2_ref, yin_ref,
                out_ref):
    del yin_ref
    _ffn_compute(x_ref, w1_ref, w2_ref, b1_ref, b2_ref, out_ref)


def _grouped_ffn(xp_seg, n_seg, off, pad_n, emap, W1, b1r, W2, b2r,
                 y_alias=None):
    d = xp_seg.shape[1]
    nf = W1.shape[2] // _F
    in_specs = [
        pl.BlockSpec((_T, d), lambda t, f, em: (t, 0)),
        pl.BlockSpec((1, d, _F), lambda t, f, em: (em[t + off], 0, f)),
        pl.BlockSpec((1, _F, d), lambda t, f, em: (em[t + off], f, 0)),
        pl.BlockSpec((1, 1, 1, _F), lambda t, f, em: (em[t + off], f, 0, 0)),
        pl.BlockSpec((1, 1, d), lambda t, f, em: (em[t + off], 0, 0)),
    ]
    body = _ffn_body_a
    args = [emap, xp_seg, W1, W2, b1r, b2r]
    aliases = {}
    if y_alias is not None:
        in_specs.append(pl.BlockSpec(memory_space=pl.ANY))
        body = _ffn_body_b
        args.append(y_alias)
        aliases = {6: 0}
    grid_spec = pltpu.PrefetchScalarGridSpec(
        num_scalar_prefetch=1,
        grid=(n_seg, nf),
        in_specs=in_specs,
        out_specs=pl.BlockSpec((_T, d), lambda t, f, em: (t + off, 0)),
    )
    return pl.pallas_call(
        body,
        grid_spec=grid_spec,
        out_shape=jax.ShapeDtypeStruct((pad_n, d), jnp.float32),
        input_output_aliases=aliases,
        compiler_params=pltpu.CompilerParams(
            dimension_semantics=("arbitrary", "arbitrary")),
    )(*args)


def _sc_gather(table, idx):
    n = idx.shape[0]
    d = table.shape[1]
    per_w = n // _NW
    nch = per_w // _CH
    mesh = plsc.VectorSubcoreMesh(core_axis_name="c", subcore_axis_name="s")

    @functools.partial(
        pl.kernel,
        mesh=mesh,
        out_type=jax.ShapeDtypeStruct((n, d), table.dtype),
        scratch_types=[
            pltpu.VMEM((per_w,), jnp.int32),
            pltpu.VMEM((2, _CH, d), table.dtype),
            pltpu.SemaphoreType.DMA,
            pltpu.SemaphoreType.DMA,
            pltpu.SemaphoreType.DMA,
            pltpu.SemaphoreType.DMA,
        ],
    )
    def gather_k(table_hbm, idx_hbm, out_hbm, idx_v, buf, g0, g1, s0, s1):
        wid = lax.axis_index("s") * 2 + lax.axis_index("c")
        base = wid * per_w
        pltpu.sync_copy(idx_hbm.at[pl.ds(base, per_w)], idx_v)
        gsem = (g0, g1)
        ssem = (s0, s1)

        def start_gather(i, p):
            pltpu.async_copy(
                table_hbm.at[idx_v.at[pl.ds(i * _CH, _CH)]], buf.at[p], gsem[p])

        def wait_gather(p):
            pltpu.make_async_copy(
                table_hbm.at[idx_v.at[pl.ds(0, _CH)]], buf.at[p],
                gsem[p]).wait()

        def start_scatter(i, p):
            pltpu.async_copy(
                buf.at[p], out_hbm.at[pl.ds(base + i * _CH, _CH)], ssem[p])

        def wait_scatter(p):
            pltpu.make_async_copy(
                buf.at[p], out_hbm.at[pl.ds(base, _CH)], ssem[p]).wait()

        start_gather(0, 0)

        def body(j, carry):
            for k in range(2):
                i = 2 * j + k
                p = k
                q = 1 - k

                @pl.when(i + 1 < nch)
                def _(i=i, p=p, q=q):
                    @pl.when(i >= 1)
                    def _():
                        wait_scatter(q)
                    start_gather(i + 1, q)

                wait_gather(p)
                start_scatter(i, p)
            return carry

        lax.fori_loop(0, nch // 2, body, 0)
        wait_scatter((nch - 2) % 2)
        wait_scatter((nch - 1) % 2)

    return gather_k(table, idx)


def kernel(inputs, dispatch_order, W1, b1, W2, b2):
    b, s, d = inputs.shape
    e, _, dff = W1.shape
    n = b * s
    flat = inputs.reshape(n, d)
    n_tiles = n // _T + e
    pad_n = n_tiles * _T

    d32 = dispatch_order.astype(jnp.int32)
    onehot = (d32[:, None] == jnp.arange(e, dtype=jnp.int32)[None, :]).astype(jnp.int32)
    counts = onehot.sum(axis=0)
    ranks = jnp.take_along_axis(jnp.cumsum(onehot, axis=0), d32[:, None], axis=1)[:, 0] - 1
    padded = ((counts + _T - 1) // _T) * _T
    csum = jnp.cumsum(padded)
    starts = csum - padded
    dest = starts[d32] + ranks
    gather_src = jnp.remainder(jnp.arange(pad_n, dtype=jnp.int32),
                               jnp.int32(n)).at[dest].set(
        jnp.arange(n, dtype=jnp.int32))
    ends_t = csum // _T
    tiles = jnp.arange(n_tiles, dtype=jnp.int32)
    emap = jnp.minimum((tiles[:, None] >= ends_t[None, :]).sum(axis=1),
                       e - 1).astype(jnp.int32)
    n_used = csum[-1] // _T
    b1r = b1.reshape(e, dff // _F, 1, _F)
    b2r = b2.reshape(e, 1, d)

    na = _TA * _T
    xp_a = _sc_gather(flat, gather_src[:na])
    xp_b = _sc_gather(flat, gather_src[na:])
    y0 = _grouped_ffn(xp_a, _TA, 0, pad_n, emap, W1, b1r, W2, b2r)
    y = _grouped_ffn(xp_b, n_used - _TA, _TA, pad_n, emap, W1, b1r, W2, b2r,
                     y_alias=y0)
    out = _sc_gather(y, dest)
    return out.reshape(b, s, d)


# speedup vs baseline: 1.0790x; 1.0790x over previous
import functools

import jax
import jax.numpy as jnp
from jax import lax
from jax.experimental import pallas as pl
from jax.experimental.pallas import tpu as pltpu
from jax.experimental.pallas import tpu_sc as plsc

_T = 640
_TA = 2
_F = 1024
_NW = 32
_CH = 16


def _ffn_compute(x_ref, w1_ref, w2_ref, b1_ref, b2_ref, out_ref):
    f = pl.program_id(1)
    nf = pl.num_programs(1)
    h = jnp.dot(x_ref[...], w1_ref[0], preferred_element_type=jnp.float32)
    h = jnp.maximum(h + b1_ref[0, 0, 0, :][None, :], 0.0)
    p = jnp.dot(h, w2_ref[0], preferred_element_type=jnp.float32)

    @pl.when(f == 0)
    def _():
        out_ref[...] = p

    @pl.when(f > 0)
    def _():
        out_ref[...] += p

    @pl.when(f == nf - 1)
    def _():
        out_ref[...] += b2_ref[0, 0, :][None, :]


def _ffn_body_a(emap_ref, x_ref, w1_ref, w2_ref, b1_ref, b2_ref, out_ref):
    _ffn_compute(x_ref, w1_ref, w2_ref, b1_ref, b2_ref, out_ref)


def _ffn_body_b(emap_ref, x_ref, w1_ref, w2_ref, b1_ref, b2_ref, yin_ref,
                out_ref):
    del yin_ref
    _ffn_compute(x_ref, w1_ref, w2_ref, b1_ref, b2_ref, out_ref)


def _grouped_ffn(xp_seg, n_seg, off, pad_n, emap, W1, b1r, W2, b2r,
                 y_alias=None):
    d = xp_seg.shape[1]
    nf = W1.shape[2] // _F
    in_specs = [
        pl.BlockSpec((_T, d), lambda t, f, em: (t, 0)),
        pl.BlockSpec((1, d, _F), lambda t, f, em: (em[t + off], 0, f)),
        pl.BlockSpec((1, _F, d), lambda t, f, em: (em[t + off], f, 0)),
        pl.BlockSpec((1, 1, 1, _F), lambda t, f, em: (em[t + off], f, 0, 0)),
        pl.BlockSpec((1, 1, d), lambda t, f, em: (em[t + off], 0, 0)),
    ]
    body = _ffn_body_a
    args = [emap, xp_seg, W1, W2, b1r, b2r]
    aliases = {}
    if y_alias is not None:
        in_specs.append(pl.BlockSpec(memory_space=pl.ANY))
        body = _ffn_body_b
        args.append(y_alias)
        aliases = {6: 0}
    grid_spec = pltpu.PrefetchScalarGridSpec(
        num_scalar_prefetch=1,
        grid=(n_seg, nf),
        in_specs=in_specs,
        out_specs=pl.BlockSpec((_T, d), lambda t, f, em: (t + off, 0)),
    )
    return pl.pallas_call(
        body,
        grid_spec=grid_spec,
        out_shape=jax.ShapeDtypeStruct((pad_n, d), jnp.float32),
        input_output_aliases=aliases,
        compiler_params=pltpu.CompilerParams(
            dimension_semantics=("arbitrary", "arbitrary")),
    )(*args)


def _sc_gather(table, idx):
    n = idx.shape[0]
    d = table.shape[1]
    per_w = n // _NW
    ch = next(c for c in (_CH, 8) if per_w % c == 0)
    nch = per_w // ch
    mesh = plsc.VectorSubcoreMesh(core_axis_name="c", subcore_axis_name="s")

    @functools.partial(
        pl.kernel,
        mesh=mesh,
        out_type=jax.ShapeDtypeStruct((n, d), table.dtype),
        scratch_types=[
            pltpu.VMEM((per_w,), jnp.int32),
            pltpu.VMEM((2, ch, d), table.dtype),
            pltpu.SemaphoreType.DMA,
            pltpu.SemaphoreType.DMA,
            pltpu.SemaphoreType.DMA,
            pltpu.SemaphoreType.DMA,
        ],
    )
    def gather_k(table_hbm, idx_hbm, out_hbm, idx_v, buf, g0, g1, s0, s1):
        wid = lax.axis_index("s") * 2 + lax.axis_index("c")
        base = wid * per_w
        pltpu.sync_copy(idx_hbm.at[pl.ds(base, per_w)], idx_v)
        gsem = (g0, g1)
        ssem = (s0, s1)

        def start_gather(i, p):
            pltpu.async_copy(
                table_hbm.at[idx_v.at[pl.ds(i * ch, ch)]], buf.at[p], gsem[p])

        def wait_gather(p):
            pltpu.make_async_copy(
                table_hbm.at[idx_v.at[pl.ds(0, ch)]], buf.at[p],
                gsem[p]).wait()

        def start_scatter(i, p):
            pltpu.async_copy(
                buf.at[p], out_hbm.at[pl.ds(base + i * ch, ch)], ssem[p])

        def wait_scatter(p):
            pltpu.make_async_copy(
                buf.at[p], out_hbm.at[pl.ds(base, ch)], ssem[p]).wait()

        start_gather(0, 0)

        def body(j, carry):
            for k in range(2):
                i = 2 * j + k
                p = k
                q = 1 - k

                @pl.when(i + 1 < nch)
                def _(i=i, p=p, q=q):
                    @pl.when(i >= 1)
                    def _():
                        wait_scatter(q)
                    start_gather(i + 1, q)

                wait_gather(p)
                start_scatter(i, p)
            return carry

        lax.fori_loop(0, nch // 2, body, 0)
        if nch % 2 == 1:
            wait_gather(0)
            start_scatter(nch - 1, 0)
        wait_scatter((nch - 2) % 2)
        wait_scatter((nch - 1) % 2)

    return gather_k(table, idx)


def kernel(inputs, dispatch_order, W1, b1, W2, b2):
    b, s, d = inputs.shape
    e, _, dff = W1.shape
    n = b * s
    flat = inputs.reshape(n, d)
    n_tiles = n // _T + e
    pad_n = n_tiles * _T

    d32 = dispatch_order.astype(jnp.int32)
    onehot = (d32[:, None] == jnp.arange(e, dtype=jnp.int32)[None, :]).astype(jnp.int32)
    counts = onehot.sum(axis=0)
    ranks = jnp.take_along_axis(jnp.cumsum(onehot, axis=0), d32[:, None], axis=1)[:, 0] - 1
    padded = ((counts + _T - 1) // _T) * _T
    csum = jnp.cumsum(padded)
    starts = csum - padded
    dest = starts[d32] + ranks
    gather_src = jnp.remainder(jnp.arange(pad_n, dtype=jnp.int32),
                               jnp.int32(n)).at[dest].set(
        jnp.arange(n, dtype=jnp.int32))
    ends_t = csum // _T
    tiles = jnp.arange(n_tiles, dtype=jnp.int32)
    emap = jnp.minimum((tiles[:, None] >= ends_t[None, :]).sum(axis=1),
                       e - 1).astype(jnp.int32)
    n_used = csum[-1] // _T
    b1r = b1.reshape(e, dff // _F, 1, _F)
    b2r = b2.reshape(e, 1, d)

    na = _TA * _T
    xp_a = _sc_gather(flat, gather_src[:na])
    xp_b = _sc_gather(flat, gather_src[na:])
    y0 = _grouped_ffn(xp_a, _TA, 0, pad_n, emap, W1, b1r, W2, b2r)
    y = _grouped_ffn(xp_b, n_used - _TA, _TA, pad_n, emap, W1, b1r, W2, b2r,
                     y_alias=y0)
    out = _sc_gather(y, dest)
    return out.reshape(b, s, d)


# speedup vs baseline: 1.1391x; 1.0557x over previous
import functools

import jax
import jax.numpy as jnp
from jax import lax
from jax.experimental import pallas as pl
from jax.experimental.pallas import tpu as pltpu
from jax.experimental.pallas import tpu_sc as plsc

_T = 576
_TA = 4
_F = 1024
_NW = 32
_CH = 16


def _ffn_compute(x_ref, w1_ref, w2_ref, b1_ref, b2_ref, out_ref):
    f = pl.program_id(1)
    nf = pl.num_programs(1)
    h = jnp.dot(x_ref[...], w1_ref[0], preferred_element_type=jnp.float32)
    h = jnp.maximum(h + b1_ref[0, 0, 0, :][None, :], 0.0)
    p = jnp.dot(h, w2_ref[0], preferred_element_type=jnp.float32)

    @pl.when(f == 0)
    def _():
        out_ref[...] = p

    @pl.when(f > 0)
    def _():
        out_ref[...] += p

    @pl.when(f == nf - 1)
    def _():
        out_ref[...] += b2_ref[0, 0, :][None, :]


def _ffn_body_a(emap_ref, x_ref, w1_ref, w2_ref, b1_ref, b2_ref, out_ref):
    _ffn_compute(x_ref, w1_ref, w2_ref, b1_ref, b2_ref, out_ref)


def _ffn_body_b(emap_ref, x_ref, w1_ref, w2_ref, b1_ref, b2_ref, yin_ref,
                out_ref):
    del yin_ref
    _ffn_compute(x_ref, w1_ref, w2_ref, b1_ref, b2_ref, out_ref)


def _grouped_ffn(xp_seg, n_seg, off, pad_n, emap, W1, b1r, W2, b2r,
                 y_alias=None):
    d = xp_seg.shape[1]
    nf = W1.shape[2] // _F
    in_specs = [
        pl.BlockSpec((_T, d), lambda t, f, em: (t, 0)),
        pl.BlockSpec((1, d, _F), lambda t, f, em: (em[t + off], 0, f)),
        pl.BlockSpec((1, _F, d), lambda t, f, em: (em[t + off], f, 0)),
        pl.BlockSpec((1, 1, 1, _F), lambda t, f, em: (em[t + off], f, 0, 0)),
        pl.BlockSpec((1, 1, d), lambda t, f, em: (em[t + off], 0, 0)),
    ]
    body = _ffn_body_a
    args = [emap, xp_seg, W1, W2, b1r, b2r]
    aliases = {}
    if y_alias is not None:
        in_specs.append(pl.BlockSpec(memory_space=pl.ANY))
        body = _ffn_body_b
        args.append(y_alias)
        aliases = {6: 0}
    grid_spec = pltpu.PrefetchScalarGridSpec(
        num_scalar_prefetch=1,
        grid=(n_seg, nf),
        in_specs=in_specs,
        out_specs=pl.BlockSpec((_T, d), lambda t, f, em: (t + off, 0)),
    )
    return pl.pallas_call(
        body,
        grid_spec=grid_spec,
        out_shape=jax.ShapeDtypeStruct((pad_n, d), jnp.float32),
        input_output_aliases=aliases,
        compiler_params=pltpu.CompilerParams(
            dimension_semantics=("arbitrary", "arbitrary")),
    )(*args)


def _sc_gather(table, idx):
    n = idx.shape[0]
    d = table.shape[1]
    per_w = n // _NW
    ch = next(c for c in (_CH, 8) if per_w % c == 0)
    nch = per_w // ch
    mesh = plsc.VectorSubcoreMesh(core_axis_name="c", subcore_axis_name="s")

    @functools.partial(
        pl.kernel,
        mesh=mesh,
        out_type=jax.ShapeDtypeStruct((n, d), table.dtype),
        scratch_types=[
            pltpu.VMEM((per_w,), jnp.int32),
            pltpu.VMEM((2, ch, d), table.dtype),
            pltpu.SemaphoreType.DMA,
            pltpu.SemaphoreType.DMA,
            pltpu.SemaphoreType.DMA,
            pltpu.SemaphoreType.DMA,
        ],
    )
    def gather_k(table_hbm, idx_hbm, out_hbm, idx_v, buf, g0, g1, s0, s1):
        wid = lax.axis_index("s") * 2 + lax.axis_index("c")
        base = wid * per_w
        pltpu.sync_copy(idx_hbm.at[pl.ds(base, per_w)], idx_v)
        gsem = (g0, g1)
        ssem = (s0, s1)

        def start_gather(i, p):
            pltpu.async_copy(
                table_hbm.at[idx_v.at[pl.ds(i * ch, ch)]], buf.at[p], gsem[p])

        def wait_gather(p):
            pltpu.make_async_copy(
                table_hbm.at[idx_v.at[pl.ds(0, ch)]], buf.at[p],
                gsem[p]).wait()

        def start_scatter(i, p):
            pltpu.async_copy(
                buf.at[p], out_hbm.at[pl.ds(base + i * ch, ch)], ssem[p])

        def wait_scatter(p):
            pltpu.make_async_copy(
                buf.at[p], out_hbm.at[pl.ds(base, ch)], ssem[p]).wait()

        start_gather(0, 0)

        def body(j, carry):
            for k in range(2):
                i = 2 * j + k
                p = k
                q = 1 - k

                @pl.when(i + 1 < nch)
                def _(i=i, p=p, q=q):
                    @pl.when(i >= 1)
                    def _():
                        wait_scatter(q)
                    start_gather(i + 1, q)

                wait_gather(p)
                start_scatter(i, p)
            return carry

        lax.fori_loop(0, nch // 2, body, 0)
        if nch % 2 == 1:
            wait_gather(0)
            start_scatter(nch - 1, 0)
        wait_scatter((nch - 2) % 2)
        wait_scatter((nch - 1) % 2)

    return gather_k(table, idx)


def kernel(inputs, dispatch_order, W1, b1, W2, b2):
    b, s, d = inputs.shape
    e, _, dff = W1.shape
    n = b * s
    flat = inputs.reshape(n, d)
    n_tiles = n // _T + e
    while (n_tiles * _T) % 256:
        n_tiles += 1
    pad_n = n_tiles * _T

    d32 = dispatch_order.astype(jnp.int32)
    onehot = (d32[:, None] == jnp.arange(e, dtype=jnp.int32)[None, :]).astype(jnp.int32)
    counts = onehot.sum(axis=0)
    ranks = jnp.take_along_axis(jnp.cumsum(onehot, axis=0), d32[:, None], axis=1)[:, 0] - 1
    padded = ((counts + _T - 1) // _T) * _T
    csum = jnp.cumsum(padded)
    starts = csum - padded
    dest = starts[d32] + ranks
    gather_src = jnp.remainder(jnp.arange(pad_n, dtype=jnp.int32),
                               jnp.int32(n)).at[dest].set(
        jnp.arange(n, dtype=jnp.int32))
    ends_t = csum // _T
    tiles = jnp.arange(n_tiles, dtype=jnp.int32)
    emap = jnp.minimum((tiles[:, None] >= ends_t[None, :]).sum(axis=1),
                       e - 1).astype(jnp.int32)
    n_used = csum[-1] // _T
    b1r = b1.reshape(e, dff // _F, 1, _F)
    b2r = b2.reshape(e, 1, d)

    na = _TA * _T
    xp_a = _sc_gather(flat, gather_src[:na])
    xp_b = _sc_gather(flat, gather_src[na:])
    y0 = _grouped_ffn(xp_a, _TA, 0, pad_n, emap, W1, b1r, W2, b2r)
    y = _grouped_ffn(xp_b, n_used - _TA, _TA, pad_n, emap, W1, b1r, W2, b2r,
                     y_alias=y0)
    out = _sc_gather(y, dest)
    return out.reshape(b, s, d)
